# baseline (speedup 1.0000x reference)
import functools

import jax
import jax.numpy as jnp
from jax import lax
from jax.experimental import pallas as pl
from jax.experimental.pallas import tpu as pltpu
from jax.experimental.pallas import tpu_sc as plsc

NC, NS, L = 2, 16, 16
NW = NC * NS
R, C = 512, 100000
RSC = 224
RTC = R - RSC
RPW = RSC // NW
HALF0 = 49920
HALF1 = C - HALF0
U = 5
STEPS0 = HALF0 // (L * U)
STEPS1 = HALF1 // (L * U)
BIG = 1 << 30

TCB = 8
CH = 1408
NCH = 71
CTAIL0 = C - 128


def _sc_body(x_hbm, tgt_hbm, m_hbm, s_hbm, mi_hbm, tgl_hbm, tv_hbm,
             buf0, buf1, tgtv, om, os_, omi, otgl, otv, sem0, sem1):
    wid = lax.axis_index("c") * NS + lax.axis_index("s")
    base = wid * RPW
    lane = lax.iota(jnp.int32, L)

    pltpu.sync_copy(tgt_hbm.at[pl.ds(0, RSC)], tgtv)
    pltpu.make_async_copy(
        x_hbm.at[base].at[pl.ds(0, HALF0)], buf0, sem0).start()

    def half_pass(buf, steps, carry):
        def body(i, c):
            ms, ss = c[:U], c[U:]
            ms2, ss2 = [], []
            for u in range(U):
                off = i * (L * U) + u * L
                v = buf[pl.ds(off, L)]
                ms2.append(jnp.maximum(ms[u], v))
                ss2.append(ss[u] + jnp.exp(v))
            return tuple(ms2) + tuple(ss2)
        return lax.fori_loop(0, steps, body, carry)

    def tgl_from(buf, t16, lo, size, tgl_prev):
        inb = (t16 >= lo) & (t16 < lo + size)
        lidx = jnp.clip(t16 - lo, 0, size - 1)
        g = plsc.load_gather(buf, [lidx])
        return jnp.where(inb, g, tgl_prev)

    def row_step(r, _):
        row = base + r
        pltpu.make_async_copy(
            x_hbm.at[row].at[pl.ds(0, HALF0)], buf0, sem0).wait()
        pltpu.make_async_copy(
            x_hbm.at[row].at[pl.ds(HALF0, HALF1)], buf1, sem1).start()
        t16 = plsc.load_gather(tgtv, [jnp.full((L,), row, jnp.int32)])

        init = tuple(jnp.full((L,), -jnp.inf, jnp.float32) for _ in range(U)) \
            + tuple(jnp.zeros((L,), jnp.float32) for _ in range(U))
        c0 = half_pass(buf0, STEPS0, init)
        tgl16 = tgl_from(buf0, t16, 0, HALF0, jnp.zeros((L,), jnp.float32))

        pltpu.make_async_copy(
            x_hbm.at[row].at[pl.ds(HALF0, HALF1)], buf1, sem1).wait()

        @pl.when(r < RPW - 1)
        def _():
            pltpu.make_async_copy(
                x_hbm.at[row + 1].at[pl.ds(0, HALF0)], buf0, sem0).start()

        c1 = half_pass(buf1, STEPS1, c0)
        tgl16 = tgl_from(buf1, t16, HALF0, HALF1, tgl16)

        ms, ss = c1[:U], c1[U:]
        m16 = ms[0]
        s16 = ss[0]
        for u in range(1, U):
            m16 = jnp.maximum(m16, ms[u])
            s16 = s16 + ss[u]

        ma = jnp.max(m16)
        tg = jnp.max(tgl16)

        def rare_scan():
            bm = jnp.full((L,), ma, jnp.float32)

            def eq_scan(buf, goff, steps, mi0):
                def rstep(i, mi):
                    for u in range(U):
                        off = i * (L * U) + u * L
                        v = buf[pl.ds(off, L)]
                        gidx = goff + off + lane
                        hit = (v == bm) & (mi == BIG)
                        mi = jnp.where(hit, gidx, mi)
                    return mi
                return lax.fori_loop(0, steps, rstep, mi0)

            @pl.when(r < RPW - 1)
            def _():
                pltpu.make_async_copy(
                    x_hbm.at[row + 1].at[pl.ds(0, HALF0)], buf0, sem0).wait()

            pltpu.sync_copy(x_hbm.at[row].at[pl.ds(0, HALF0)], buf0)
            mi16 = eq_scan(buf0, 0, STEPS0, jnp.full((L,), BIG, jnp.int32))
            mi16 = eq_scan(buf1, HALF0, STEPS1, mi16)

            @pl.when(r < RPW - 1)
            def _():
                pltpu.make_async_copy(
                    x_hbm.at[row + 1].at[pl.ds(0, HALF0)], buf0, sem0).start()

            return mi16

        mi16 = lax.cond(tg == ma, rare_scan,
                        lambda: jnp.full((L,), BIG, jnp.int32))

        om[pl.ds(r * L, L)] = m16
        os_[pl.ds(r * L, L)] = s16
        omi[pl.ds(r * L, L)] = mi16
        otgl[pl.ds(r * L, L)] = tgl16
        otv[pl.ds(r * L, L)] = t16
        return 0

    lax.fori_loop(0, RPW, row_step, 0)

    fb = base * L
    pltpu.sync_copy(om, m_hbm.at[pl.ds(fb, RPW * L)])
    pltpu.sync_copy(os_, s_hbm.at[pl.ds(fb, RPW * L)])
    pltpu.sync_copy(omi, mi_hbm.at[pl.ds(fb, RPW * L)])
    pltpu.sync_copy(otgl, tgl_hbm.at[pl.ds(fb, RPW * L)])
    pltpu.sync_copy(otv, tv_hbm.at[pl.ds(fb, RPW * L)])


_sc_call = functools.partial(
    pl.kernel,
    out_type=(jax.ShapeDtypeStruct((RSC * L,), jnp.float32),
              jax.ShapeDtypeStruct((RSC * L,), jnp.float32),
              jax.ShapeDtypeStruct((RSC * L,), jnp.int32),
              jax.ShapeDtypeStruct((RSC * L,), jnp.float32),
              jax.ShapeDtypeStruct((RSC * L,), jnp.int32)),
    mesh=plsc.VectorSubcoreMesh(
        core_axis_name="c", subcore_axis_name="s",
        num_cores=NC, num_subcores=NS),
    compiler_params=pltpu.CompilerParams(needs_layout_passes=False),
    scratch_types=[
        pltpu.VMEM((HALF0,), jnp.float32),
        pltpu.VMEM((HALF1,), jnp.float32),
        pltpu.VMEM((RSC,), jnp.int32),
        pltpu.VMEM((RPW * L,), jnp.float32),
        pltpu.VMEM((RPW * L,), jnp.float32),
        pltpu.VMEM((RPW * L,), jnp.int32),
        pltpu.VMEM((RPW * L,), jnp.float32),
        pltpu.VMEM((RPW * L,), jnp.int32),
        pltpu.SemaphoreType.DMA,
        pltpu.SemaphoreType.DMA,
    ],
)(_sc_body)


def _tc_body(tgt_ref, x_ref, o_ref):
    pid = pl.program_id(0)
    iota = lax.broadcasted_iota(jnp.int32, (TCB, 128), 1)

    def tree(op, xs):
        while len(xs) > 1:
            nxt = [op(xs[k], xs[k + 1]) for k in range(0, len(xs) - 1, 2)]
            if len(xs) % 2:
                nxt.append(xs[-1])
            xs = nxt
        return xs[0]

    def fold(base, width, m, s):
        v = x_ref[:, pl.ds(base, width)]
        vs = [v[:, k * 128:(k + 1) * 128] for k in range(width // 128)]
        m = jnp.maximum(m, tree(jnp.maximum, vs))
        s = s + tree(jnp.add, [jnp.exp(u) for u in vs])
        return m, s

    def chunk(i, carry):
        return fold(i * (2 * CH), 2 * CH, *carry)

    m, s = lax.fori_loop(
        0, NCH // 2, chunk,
        (jnp.full((TCB, 128), -jnp.inf, jnp.float32),
         jnp.zeros((TCB, 128), jnp.float32)))
    m, s = fold((NCH - 1) * CH, CH, m, s)

    v = x_ref[:, pl.ds(CTAIL0, 128)]
    live = iota >= 96
    s = s + jnp.where(live, jnp.exp(v), 0.0)
    m = jnp.maximum(m, jnp.where(live, v, -jnp.inf))

    mrow = jnp.max(m, axis=1, keepdims=True)
    srow = jnp.sum(s, axis=1, keepdims=True)

    tg_scalars = [tgt_ref[RSC + pid * TCB + r] for r in range(TCB)]
    lane1 = lax.broadcasted_iota(jnp.int32, (1, 128), 1)

    def _pick(r):
        t = tg_scalars[r]
        off = pl.multiple_of((t >> 7) << 7, 128)
        v = x_ref[pl.ds(r, 1), pl.ds(off, 128)]
        return jnp.sum(jnp.where(lane1 == (t & 127), v, 0.0),
                       axis=1, keepdims=True)

    tglv = jnp.concatenate([_pick(r) for r in range(TCB)], axis=0)
    trow = jnp.concatenate(
        [jnp.full((1, 1), t, jnp.int32) for t in tg_scalars], axis=0)

    anytie = jnp.any(tglv == mrow)

    def tie_scan():
        def body(i, mi):
            base = i * CH
            for j in range(11):
                v = x_ref[:, pl.ds(base + j * 128, 128)]
                colv = iota + (base + j * 128)
                hit = jnp.logical_and(v == mrow, colv < mi)
                mi = jnp.where(hit, colv, mi)
            return mi
        mi = lax.fori_loop(0, NCH, body,
                           jnp.full((TCB, 128), BIG, jnp.int32))
        v = x_ref[:, pl.ds(CTAIL0, 128)]
        colv = iota + CTAIL0
        hit = jnp.logical_and(jnp.logical_and(live, v == mrow), colv < mi)
        mi = jnp.where(hit, colv, mi)
        return jnp.min(mi, axis=1, keepdims=True)

    mirow = lax.cond(anytie, tie_scan,
                     lambda: jnp.full((TCB, 1), BIG, jnp.int32))

    correct = jnp.logical_and(tglv == mrow, mirow == trow)
    wrong = 1.0 - correct.astype(jnp.float32)
    bsum = jnp.sum((jnp.log(srow) - tglv) * wrong)

    @pl.when(pid == 0)
    def _():
        o_ref[0, 0] = 0.0
    o_ref[0, 0] += bsum


_tc_partial = pl.pallas_call(
    _tc_body,
    grid_spec=pltpu.PrefetchScalarGridSpec(
        num_scalar_prefetch=1,
        grid=(RTC // TCB,),
        in_specs=[pl.BlockSpec((TCB, C), lambda i, *_: (RSC // TCB + i, 0))],
        out_specs=pl.BlockSpec(memory_space=pltpu.SMEM),
    ),
    out_shape=jax.ShapeDtypeStruct((1, 1), jnp.float32),
)


def _combine_body(m_ref, s_ref, mi_ref, tgl_ref, tv_ref, tc_ref, o_ref):
    m = m_ref[...]
    mrow = jnp.max(m, axis=1, keepdims=True)
    srow = jnp.sum(s_ref[...], axis=1, keepdims=True)
    mifirst = jnp.min(mi_ref[...], axis=1, keepdims=True)
    tgl = tgl_ref[:, 0:1]
    tv = tv_ref[:, 0:1]
    correct = jnp.logical_and(tgl == mrow, mifirst == tv)
    wrong = 1.0 - correct.astype(jnp.float32)
    loss = (jnp.log(srow) - tgl) * wrong
    o_ref[0, 0] = (jnp.sum(loss) + tc_ref[0, 0]) / jnp.float32(R)


_combine = pl.pallas_call(
    _combine_body,
    out_shape=jax.ShapeDtypeStruct((1, 1), jnp.float32),
    in_specs=[pl.BlockSpec(memory_space=pltpu.VMEM)] * 5
    + [pl.BlockSpec(memory_space=pltpu.SMEM)],
    out_specs=pl.BlockSpec(memory_space=pltpu.SMEM),
)


def kernel(output, target):
    x = output.reshape(R, C)
    t = target.reshape(R).astype(jnp.int32)
    m, s, mi, tgl, tv = _sc_call(x, t)
    tc_sum = _tc_partial(t, x)
    return _combine(m.reshape(RSC, L), s.reshape(RSC, L),
                    mi.reshape(RSC, L), tgl.reshape(RSC, L),
                    tv.reshape(RSC, L), tc_sum).reshape(())


# speedup vs baseline: 1.0609x; 1.0609x over previous
import functools

import jax
import jax.numpy as jnp
from jax import lax
from jax.experimental import pallas as pl
from jax.experimental.pallas import tpu as pltpu
from jax.experimental.pallas import tpu_sc as plsc

NC, NS, L = 2, 16, 16
NW = NC * NS
R, C = 512, 100000
RSC = 256
RTC = R - RSC
RPW = RSC // NW
HALF0 = 49920
HALF1 = C - HALF0
U = 5
STEPS0 = HALF0 // (L * U)
STEPS1 = HALF1 // (L * U)
BIG = 1 << 30

TCB = 16
CH = 1408
NCH = 71
CTAIL0 = C - 128


def _sc_body(x_hbm, tgt_hbm, m_hbm, s_hbm, mi_hbm, tgl_hbm, tv_hbm,
             buf0, buf1, tgtv, om, os_, omi, otgl, otv, sem0, sem1):
    wid = lax.axis_index("c") * NS + lax.axis_index("s")
    base = wid * RPW
    lane = lax.iota(jnp.int32, L)

    pltpu.sync_copy(tgt_hbm.at[pl.ds(0, RSC)], tgtv)
    pltpu.make_async_copy(
        x_hbm.at[base].at[pl.ds(0, HALF0)], buf0, sem0).start()

    def half_pass(buf, steps, carry):
        def body(i, c):
            ms, ss = c[:U], c[U:]
            ms2, ss2 = [], []
            for u in range(U):
                off = i * (L * U) + u * L
                v = buf[pl.ds(off, L)]
                ms2.append(jnp.maximum(ms[u], v))
                ss2.append(ss[u] + jnp.exp(v))
            return tuple(ms2) + tuple(ss2)
        return lax.fori_loop(0, steps, body, carry)

    def tgl_from(buf, t16, lo, size, tgl_prev):
        inb = (t16 >= lo) & (t16 < lo + size)
        lidx = jnp.clip(t16 - lo, 0, size - 1)
        g = plsc.load_gather(buf, [lidx])
        return jnp.where(inb, g, tgl_prev)

    def row_step(r, _):
        row = base + r
        pltpu.make_async_copy(
            x_hbm.at[row].at[pl.ds(0, HALF0)], buf0, sem0).wait()
        pltpu.make_async_copy(
            x_hbm.at[row].at[pl.ds(HALF0, HALF1)], buf1, sem1).start()
        t16 = plsc.load_gather(tgtv, [jnp.full((L,), row, jnp.int32)])

        init = tuple(jnp.full((L,), -jnp.inf, jnp.float32) for _ in range(U)) \
            + tuple(jnp.zeros((L,), jnp.float32) for _ in range(U))
        c0 = half_pass(buf0, STEPS0, init)
        tgl16 = tgl_from(buf0, t16, 0, HALF0, jnp.zeros((L,), jnp.float32))

        pltpu.make_async_copy(
            x_hbm.at[row].at[pl.ds(HALF0, HALF1)], buf1, sem1).wait()

        @pl.when(r < RPW - 1)
        def _():
            pltpu.make_async_copy(
                x_hbm.at[row + 1].at[pl.ds(0, HALF0)], buf0, sem0).start()

        c1 = half_pass(buf1, STEPS1, c0)
        tgl16 = tgl_from(buf1, t16, HALF0, HALF1, tgl16)

        ms, ss = c1[:U], c1[U:]
        m16 = ms[0]
        s16 = ss[0]
        for u in range(1, U):
            m16 = jnp.maximum(m16, ms[u])
            s16 = s16 + ss[u]

        ma = jnp.max(m16)
        tg = jnp.max(tgl16)

        def rare_scan():
            bm = jnp.full((L,), ma, jnp.float32)

            def eq_scan(buf, goff, steps, mi0):
                def rstep(i, mi):
                    for u in range(U):
                        off = i * (L * U) + u * L
                        v = buf[pl.ds(off, L)]
                        gidx = goff + off + lane
                        hit = (v == bm) & (mi == BIG)
                        mi = jnp.where(hit, gidx, mi)
                    return mi
                return lax.fori_loop(0, steps, rstep, mi0)

            @pl.when(r < RPW - 1)
            def _():
                pltpu.make_async_copy(
                    x_hbm.at[row + 1].at[pl.ds(0, HALF0)], buf0, sem0).wait()

            pltpu.sync_copy(x_hbm.at[row].at[pl.ds(0, HALF0)], buf0)
            mi16 = eq_scan(buf0, 0, STEPS0, jnp.full((L,), BIG, jnp.int32))
            mi16 = eq_scan(buf1, HALF0, STEPS1, mi16)

            @pl.when(r < RPW - 1)
            def _():
                pltpu.make_async_copy(
                    x_hbm.at[row + 1].at[pl.ds(0, HALF0)], buf0, sem0).start()

            return mi16

        mi16 = lax.cond(tg == ma, rare_scan,
                        lambda: jnp.full((L,), BIG, jnp.int32))

        om[pl.ds(r * L, L)] = m16
        os_[pl.ds(r * L, L)] = s16
        omi[pl.ds(r * L, L)] = mi16
        otgl[pl.ds(r * L, L)] = tgl16
        otv[pl.ds(r * L, L)] = t16
        return 0

    lax.fori_loop(0, RPW, row_step, 0)

    fb = base * L
    pltpu.sync_copy(om, m_hbm.at[pl.ds(fb, RPW * L)])
    pltpu.sync_copy(os_, s_hbm.at[pl.ds(fb, RPW * L)])
    pltpu.sync_copy(omi, mi_hbm.at[pl.ds(fb, RPW * L)])
    pltpu.sync_copy(otgl, tgl_hbm.at[pl.ds(fb, RPW * L)])
    pltpu.sync_copy(otv, tv_hbm.at[pl.ds(fb, RPW * L)])


_sc_call = functools.partial(
    pl.kernel,
    out_type=(jax.ShapeDtypeStruct((RSC * L,), jnp.float32),
              jax.ShapeDtypeStruct((RSC * L,), jnp.float32),
              jax.ShapeDtypeStruct((RSC * L,), jnp.int32),
              jax.ShapeDtypeStruct((RSC * L,), jnp.float32),
              jax.ShapeDtypeStruct((RSC * L,), jnp.int32)),
    mesh=plsc.VectorSubcoreMesh(
        core_axis_name="c", subcore_axis_name="s",
        num_cores=NC, num_subcores=NS),
    compiler_params=pltpu.CompilerParams(needs_layout_passes=False),
    scratch_types=[
        pltpu.VMEM((HALF0,), jnp.float32),
        pltpu.VMEM((HALF1,), jnp.float32),
        pltpu.VMEM((RSC,), jnp.int32),
        pltpu.VMEM((RPW * L,), jnp.float32),
        pltpu.VMEM((RPW * L,), jnp.float32),
        pltpu.VMEM((RPW * L,), jnp.int32),
        pltpu.VMEM((RPW * L,), jnp.float32),
        pltpu.VMEM((RPW * L,), jnp.int32),
        pltpu.SemaphoreType.DMA,
        pltpu.SemaphoreType.DMA,
    ],
)(_sc_body)


def _tc_body(tgt_ref, x_ref, o_ref):
    pid = pl.program_id(0)
    iota = lax.broadcasted_iota(jnp.int32, (TCB, 128), 1)

    def tree(op, xs):
        while len(xs) > 1:
            nxt = [op(xs[k], xs[k + 1]) for k in range(0, len(xs) - 1, 2)]
            if len(xs) % 2:
                nxt.append(xs[-1])
            xs = nxt
        return xs[0]

    def fold(base, width, m, s):
        v = x_ref[:, pl.ds(base, width)]
        vs = [v[:, k * 128:(k + 1) * 128] for k in range(width // 128)]
        m = jnp.maximum(m, tree(jnp.maximum, vs))
        s = s + tree(jnp.add, [jnp.exp(u) for u in vs])
        return m, s

    def chunk(i, carry):
        return fold(i * (2 * CH), 2 * CH, *carry)

    m, s = lax.fori_loop(
        0, NCH // 2, chunk,
        (jnp.full((TCB, 128), -jnp.inf, jnp.float32),
         jnp.zeros((TCB, 128), jnp.float32)))
    m, s = fold((NCH - 1) * CH, CH, m, s)

    v = x_ref[:, pl.ds(CTAIL0, 128)]
    live = iota >= 96
    s = s + jnp.where(live, jnp.exp(v), 0.0)
    m = jnp.maximum(m, jnp.where(live, v, -jnp.inf))

    mrow = jnp.max(m, axis=1, keepdims=True)
    srow = jnp.sum(s, axis=1, keepdims=True)

    tg_scalars = [tgt_ref[RSC + pid * TCB + r] for r in range(TCB)]
    lane1 = lax.broadcasted_iota(jnp.int32, (1, 128), 1)

    def _pick(r):
        t = tg_scalars[r]
        off = pl.multiple_of((t >> 7) << 7, 128)
        v = x_ref[pl.ds(r, 1), pl.ds(off, 128)]
        return jnp.sum(jnp.where(lane1 == (t & 127), v, 0.0),
                       axis=1, keepdims=True)

    tglv = jnp.concatenate([_pick(r) for r in range(TCB)], axis=0)
    trow = jnp.concatenate(
        [jnp.full((1, 1), t, jnp.int32) for t in tg_scalars], axis=0)

    anytie = jnp.any(tglv == mrow)

    def tie_scan():
        def body(i, mi):
            base = i * CH
            for j in range(11):
                v = x_ref[:, pl.ds(base + j * 128, 128)]
                colv = iota + (base + j * 128)
                hit = jnp.logical_and(v == mrow, colv < mi)
                mi = jnp.where(hit, colv, mi)
            return mi
        mi = lax.fori_loop(0, NCH, body,
                           jnp.full((TCB, 128), BIG, jnp.int32))
        v = x_ref[:, pl.ds(CTAIL0, 128)]
        colv = iota + CTAIL0
        hit = jnp.logical_and(jnp.logical_and(live, v == mrow), colv < mi)
        mi = jnp.where(hit, colv, mi)
        return jnp.min(mi, axis=1, keepdims=True)

    mirow = lax.cond(anytie, tie_scan,
                     lambda: jnp.full((TCB, 1), BIG, jnp.int32))

    correct = jnp.logical_and(tglv == mrow, mirow == trow)
    wrong = 1.0 - correct.astype(jnp.float32)
    bsum = jnp.sum((jnp.log(srow) - tglv) * wrong)

    @pl.when(pid == 0)
    def _():
        o_ref[0, 0] = 0.0
    o_ref[0, 0] += bsum


_tc_partial = pl.pallas_call(
    _tc_body,
    grid_spec=pltpu.PrefetchScalarGridSpec(
        num_scalar_prefetch=1,
        grid=(RTC // TCB,),
        in_specs=[pl.BlockSpec((TCB, C), lambda i, *_: (RSC // TCB + i, 0))],
        out_specs=pl.BlockSpec(memory_space=pltpu.SMEM),
    ),
    out_shape=jax.ShapeDtypeStruct((1, 1), jnp.float32),
)


def _combine_body(m_ref, s_ref, mi_ref, tgl_ref, tv_ref, tc_ref, o_ref):
    m = m_ref[...]
    mrow = jnp.max(m, axis=1, keepdims=True)
    srow = jnp.sum(s_ref[...], axis=1, keepdims=True)
    mifirst = jnp.min(mi_ref[...], axis=1, keepdims=True)
    tgl = tgl_ref[:, 0:1]
    tv = tv_ref[:, 0:1]
    correct = jnp.logical_and(tgl == mrow, mifirst == tv)
    wrong = 1.0 - correct.astype(jnp.float32)
    loss = (jnp.log(srow) - tgl) * wrong
    o_ref[0, 0] = (jnp.sum(loss) + tc_ref[0, 0]) / jnp.float32(R)


_combine = pl.pallas_call(
    _combine_body,
    out_shape=jax.ShapeDtypeStruct((1, 1), jnp.float32),
    in_specs=[pl.BlockSpec(memory_space=pltpu.VMEM)] * 5
    + [pl.BlockSpec(memory_space=pltpu.SMEM)],
    out_specs=pl.BlockSpec(memory_space=pltpu.SMEM),
)


def kernel(output, target):
    x = output.reshape(R, C)
    t = target.reshape(R).astype(jnp.int32)
    m, s, mi, tgl, tv = _sc_call(x, t)
    tc_sum = _tc_partial(t, x)
    return _combine(m.reshape(RSC, L), s.reshape(RSC, L),
                    mi.reshape(RSC, L), tgl.reshape(RSC, L),
                    tv.reshape(RSC, L), tc_sum).reshape(())


# speedup vs baseline: 1.0654x; 1.0043x over previous
import functools

import jax
import jax.numpy as jnp
from jax import lax
from jax.experimental import pallas as pl
from jax.experimental.pallas import tpu as pltpu
from jax.experimental.pallas import tpu_sc as plsc

NC, NS, L = 2, 16, 16
NW = NC * NS
R, C = 512, 100000
RSC = 256
RTC = R - RSC
RPW = RSC // NW
HALF0 = 49920
HALF1 = C - HALF0
U = 5
STEPS0 = HALF0 // (L * U)
STEPS1 = HALF1 // (L * U)
BIG = 1 << 30

TCB = 32
CH = 1408
NCH = 71
CTAIL0 = C - 128


def _sc_body(x_hbm, tgt_hbm, m_hbm, s_hbm, mi_hbm, tgl_hbm, tv_hbm,
             buf0, buf1, tgtv, om, os_, omi, otgl, otv, sem0, sem1):
    wid = lax.axis_index("c") * NS + lax.axis_index("s")
    base = wid * RPW
    lane = lax.iota(jnp.int32, L)

    pltpu.sync_copy(tgt_hbm.at[pl.ds(0, RSC)], tgtv)
    pltpu.make_async_copy(
        x_hbm.at[base].at[pl.ds(0, HALF0)], buf0, sem0).start()

    def half_pass(buf, steps, carry):
        def body(i, c):
            ms, ss = c[:U], c[U:]
            ms2, ss2 = [], []
            for u in range(U):
                off = i * (L * U) + u * L
                v = buf[pl.ds(off, L)]
                ms2.append(jnp.maximum(ms[u], v))
                ss2.append(ss[u] + jnp.exp(v))
            return tuple(ms2) + tuple(ss2)
        return lax.fori_loop(0, steps, body, carry)

    def tgl_from(buf, t16, lo, size, tgl_prev):
        inb = (t16 >= lo) & (t16 < lo + size)
        lidx = jnp.clip(t16 - lo, 0, size - 1)
        g = plsc.load_gather(buf, [lidx])
        return jnp.where(inb, g, tgl_prev)

    def row_step(r, _):
        row = base + r
        pltpu.make_async_copy(
            x_hbm.at[row].at[pl.ds(0, HALF0)], buf0, sem0).wait()
        pltpu.make_async_copy(
            x_hbm.at[row].at[pl.ds(HALF0, HALF1)], buf1, sem1).start()
        t16 = plsc.load_gather(tgtv, [jnp.full((L,), row, jnp.int32)])

        init = tuple(jnp.full((L,), -jnp.inf, jnp.float32) for _ in range(U)) \
            + tuple(jnp.zeros((L,), jnp.float32) for _ in range(U))
        c0 = half_pass(buf0, STEPS0, init)
        tgl16 = tgl_from(buf0, t16, 0, HALF0, jnp.zeros((L,), jnp.float32))

        pltpu.make_async_copy(
            x_hbm.at[row].at[pl.ds(HALF0, HALF1)], buf1, sem1).wait()

        @pl.when(r < RPW - 1)
        def _():
            pltpu.make_async_copy(
                x_hbm.at[row + 1].at[pl.ds(0, HALF0)], buf0, sem0).start()

        c1 = half_pass(buf1, STEPS1, c0)
        tgl16 = tgl_from(buf1, t16, HALF0, HALF1, tgl16)

        ms, ss = c1[:U], c1[U:]
        m16 = ms[0]
        s16 = ss[0]
        for u in range(1, U):
            m16 = jnp.maximum(m16, ms[u])
            s16 = s16 + ss[u]

        ma = jnp.max(m16)
        tg = jnp.max(tgl16)

        def rare_scan():
            bm = jnp.full((L,), ma, jnp.float32)

            def eq_scan(buf, goff, steps, mi0):
                def rstep(i, mi):
                    for u in range(U):
                        off = i * (L * U) + u * L
                        v = buf[pl.ds(off, L)]
                        gidx = goff + off + lane
                        hit = (v == bm) & (mi == BIG)
                        mi = jnp.where(hit, gidx, mi)
                    return mi
                return lax.fori_loop(0, steps, rstep, mi0)

            @pl.when(r < RPW - 1)
            def _():
                pltpu.make_async_copy(
                    x_hbm.at[row + 1].at[pl.ds(0, HALF0)], buf0, sem0).wait()

            pltpu.sync_copy(x_hbm.at[row].at[pl.ds(0, HALF0)], buf0)
            mi16 = eq_scan(buf0, 0, STEPS0, jnp.full((L,), BIG, jnp.int32))
            mi16 = eq_scan(buf1, HALF0, STEPS1, mi16)

            @pl.when(r < RPW - 1)
            def _():
                pltpu.make_async_copy(
                    x_hbm.at[row + 1].at[pl.ds(0, HALF0)], buf0, sem0).start()

            return mi16

        mi16 = lax.cond(tg == ma, rare_scan,
                        lambda: jnp.full((L,), BIG, jnp.int32))

        om[pl.ds(r * L, L)] = m16
        os_[pl.ds(r * L, L)] = s16
        omi[pl.ds(r * L, L)] = mi16
        otgl[pl.ds(r * L, L)] = tgl16
        otv[pl.ds(r * L, L)] = t16
        return 0

    lax.fori_loop(0, RPW, row_step, 0)

    fb = base * L
    pltpu.sync_copy(om, m_hbm.at[pl.ds(fb, RPW * L)])
    pltpu.sync_copy(os_, s_hbm.at[pl.ds(fb, RPW * L)])
    pltpu.sync_copy(omi, mi_hbm.at[pl.ds(fb, RPW * L)])
    pltpu.sync_copy(otgl, tgl_hbm.at[pl.ds(fb, RPW * L)])
    pltpu.sync_copy(otv, tv_hbm.at[pl.ds(fb, RPW * L)])


_sc_call = functools.partial(
    pl.kernel,
    out_type=(jax.ShapeDtypeStruct((RSC * L,), jnp.float32),
              jax.ShapeDtypeStruct((RSC * L,), jnp.float32),
              jax.ShapeDtypeStruct((RSC * L,), jnp.int32),
              jax.ShapeDtypeStruct((RSC * L,), jnp.float32),
              jax.ShapeDtypeStruct((RSC * L,), jnp.int32)),
    mesh=plsc.VectorSubcoreMesh(
        core_axis_name="c", subcore_axis_name="s",
        num_cores=NC, num_subcores=NS),
    compiler_params=pltpu.CompilerParams(needs_layout_passes=False),
    scratch_types=[
        pltpu.VMEM((HALF0,), jnp.float32),
        pltpu.VMEM((HALF1,), jnp.float32),
        pltpu.VMEM((RSC,), jnp.int32),
        pltpu.VMEM((RPW * L,), jnp.float32),
        pltpu.VMEM((RPW * L,), jnp.float32),
        pltpu.VMEM((RPW * L,), jnp.int32),
        pltpu.VMEM((RPW * L,), jnp.float32),
        pltpu.VMEM((RPW * L,), jnp.int32),
        pltpu.SemaphoreType.DMA,
        pltpu.SemaphoreType.DMA,
    ],
)(_sc_body)


def _tc_body(tgt_ref, x_ref, o_ref):
    pid = pl.program_id(0)
    iota = lax.broadcasted_iota(jnp.int32, (TCB, 128), 1)

    def tree(op, xs):
        while len(xs) > 1:
            nxt = [op(xs[k], xs[k + 1]) for k in range(0, len(xs) - 1, 2)]
            if len(xs) % 2:
                nxt.append(xs[-1])
            xs = nxt
        return xs[0]

    def fold(base, width, m, s):
        v = x_ref[:, pl.ds(base, width)]
        vs = [v[:, k * 128:(k + 1) * 128] for k in range(width // 128)]
        m = jnp.maximum(m, tree(jnp.maximum, vs))
        s = s + tree(jnp.add, [jnp.exp(u) for u in vs])
        return m, s

    def chunk(i, carry):
        return fold(i * (2 * CH), 2 * CH, *carry)

    m, s = lax.fori_loop(
        0, NCH // 2, chunk,
        (jnp.full((TCB, 128), -jnp.inf, jnp.float32),
         jnp.zeros((TCB, 128), jnp.float32)))
    m, s = fold((NCH - 1) * CH, CH, m, s)

    v = x_ref[:, pl.ds(CTAIL0, 128)]
    live = iota >= 96
    s = s + jnp.where(live, jnp.exp(v), 0.0)
    m = jnp.maximum(m, jnp.where(live, v, -jnp.inf))

    mrow = jnp.max(m, axis=1, keepdims=True)
    srow = jnp.sum(s, axis=1, keepdims=True)

    tg_scalars = [tgt_ref[RSC + pid * TCB + r] for r in range(TCB)]
    lane1 = lax.broadcasted_iota(jnp.int32, (1, 128), 1)

    def _pick(r):
        t = tg_scalars[r]
        off = pl.multiple_of((t >> 7) << 7, 128)
        v = x_ref[pl.ds(r, 1), pl.ds(off, 128)]
        return jnp.sum(jnp.where(lane1 == (t & 127), v, 0.0),
                       axis=1, keepdims=True)

    tglv = jnp.concatenate([_pick(r) for r in range(TCB)], axis=0)
    trow = jnp.concatenate(
        [jnp.full((1, 1), t, jnp.int32) for t in tg_scalars], axis=0)

    anytie = jnp.any(tglv == mrow)

    def tie_scan():
        def body(i, mi):
            base = i * CH
            for j in range(11):
                v = x_ref[:, pl.ds(base + j * 128, 128)]
                colv = iota + (base + j * 128)
                hit = jnp.logical_and(v == mrow, colv < mi)
                mi = jnp.where(hit, colv, mi)
            return mi
        mi = lax.fori_loop(0, NCH, body,
                           jnp.full((TCB, 128), BIG, jnp.int32))
        v = x_ref[:, pl.ds(CTAIL0, 128)]
        colv = iota + CTAIL0
        hit = jnp.logical_and(jnp.logical_and(live, v == mrow), colv < mi)
        mi = jnp.where(hit, colv, mi)
        return jnp.min(mi, axis=1, keepdims=True)

    mirow = lax.cond(anytie, tie_scan,
                     lambda: jnp.full((TCB, 1), BIG, jnp.int32))

    correct = jnp.logical_and(tglv == mrow, mirow == trow)
    wrong = 1.0 - correct.astype(jnp.float32)
    bsum = jnp.sum((jnp.log(srow) - tglv) * wrong)

    @pl.when(pid == 0)
    def _():
        o_ref[0, 0] = 0.0
    o_ref[0, 0] += bsum


_tc_partial = pl.pallas_call(
    _tc_body,
    grid_spec=pltpu.PrefetchScalarGridSpec(
        num_scalar_prefetch=1,
        grid=(RTC // TCB,),
        in_specs=[pl.BlockSpec((TCB, C), lambda i, *_: (RSC // TCB + i, 0))],
        out_specs=pl.BlockSpec(memory_space=pltpu.SMEM),
    ),
    out_shape=jax.ShapeDtypeStruct((1, 1), jnp.float32),
)


def _combine_body(m_ref, s_ref, mi_ref, tgl_ref, tv_ref, tc_ref, o_ref):
    m = m_ref[...]
    mrow = jnp.max(m, axis=1, keepdims=True)
    srow = jnp.sum(s_ref[...], axis=1, keepdims=True)
    mifirst = jnp.min(mi_ref[...], axis=1, keepdims=True)
    tgl = tgl_ref[:, 0:1]
    tv = tv_ref[:, 0:1]
    correct = jnp.logical_and(tgl == mrow, mifirst == tv)
    wrong = 1.0 - correct.astype(jnp.float32)
    loss = (jnp.log(srow) - tgl) * wrong
    o_ref[0, 0] = (jnp.sum(loss) + tc_ref[0, 0]) / jnp.float32(R)


_combine = pl.pallas_call(
    _combine_body,
    out_shape=jax.ShapeDtypeStruct((1, 1), jnp.float32),
    in_specs=[pl.BlockSpec(memory_space=pltpu.VMEM)] * 5
    + [pl.BlockSpec(memory_space=pltpu.SMEM)],
    out_specs=pl.BlockSpec(memory_space=pltpu.SMEM),
)


def kernel(output, target):
    x = output.reshape(R, C)
    t = target.reshape(R).astype(jnp.int32)
    m, s, mi, tgl, tv = _sc_call(x, t)
    tc_sum = _tc_partial(t, x)
    return _combine(m.reshape(RSC, L), s.reshape(RSC, L),
                    mi.reshape(RSC, L), tgl.reshape(RSC, L),
                    tv.reshape(RSC, L), tc_sum).reshape(())


# speedup vs baseline: 1.0659x; 1.0004x over previous
import functools

import jax
import jax.numpy as jnp
from jax import lax
from jax.experimental import pallas as pl
from jax.experimental.pallas import tpu as pltpu
from jax.experimental.pallas import tpu_sc as plsc

NC, NS, L = 2, 16, 16
NW = NC * NS
R, C = 512, 100000
RSC = 256
RTC = R - RSC
RPW = RSC // NW
HALF0 = 49920
HALF1 = C - HALF0
U = 5
STEPS0 = HALF0 // (L * U)
STEPS1 = HALF1 // (L * U)
BIG = 1 << 30

TCB = 64
CH = 1408
NCH = 71
CTAIL0 = C - 128


def _sc_body(x_hbm, tgt_hbm, m_hbm, s_hbm, mi_hbm, tgl_hbm, tv_hbm,
             buf0, buf1, tgtv, om, os_, omi, otgl, otv, sem0, sem1):
    wid = lax.axis_index("c") * NS + lax.axis_index("s")
    base = wid * RPW
    lane = lax.iota(jnp.int32, L)

    pltpu.sync_copy(tgt_hbm.at[pl.ds(0, RSC)], tgtv)
    pltpu.make_async_copy(
        x_hbm.at[base].at[pl.ds(0, HALF0)], buf0, sem0).start()

    def half_pass(buf, steps, carry):
        def body(i, c):
            ms, ss = c[:U], c[U:]
            ms2, ss2 = [], []
            for u in range(U):
                off = i * (L * U) + u * L
                v = buf[pl.ds(off, L)]
                ms2.append(jnp.maximum(ms[u], v))
                ss2.append(ss[u] + jnp.exp(v))
            return tuple(ms2) + tuple(ss2)
        return lax.fori_loop(0, steps, body, carry)

    def tgl_from(buf, t16, lo, size, tgl_prev):
        inb = (t16 >= lo) & (t16 < lo + size)
        lidx = jnp.clip(t16 - lo, 0, size - 1)
        g = plsc.load_gather(buf, [lidx])
        return jnp.where(inb, g, tgl_prev)

    def row_step(r, _):
        row = base + r
        pltpu.make_async_copy(
            x_hbm.at[row].at[pl.ds(0, HALF0)], buf0, sem0).wait()
        pltpu.make_async_copy(
            x_hbm.at[row].at[pl.ds(HALF0, HALF1)], buf1, sem1).start()
        t16 = plsc.load_gather(tgtv, [jnp.full((L,), row, jnp.int32)])

        init = tuple(jnp.full((L,), -jnp.inf, jnp.float32) for _ in range(U)) \
            + tuple(jnp.zeros((L,), jnp.float32) for _ in range(U))
        c0 = half_pass(buf0, STEPS0, init)
        tgl16 = tgl_from(buf0, t16, 0, HALF0, jnp.zeros((L,), jnp.float32))

        pltpu.make_async_copy(
            x_hbm.at[row].at[pl.ds(HALF0, HALF1)], buf1, sem1).wait()

        @pl.when(r < RPW - 1)
        def _():
            pltpu.make_async_copy(
                x_hbm.at[row + 1].at[pl.ds(0, HALF0)], buf0, sem0).start()

        c1 = half_pass(buf1, STEPS1, c0)
        tgl16 = tgl_from(buf1, t16, HALF0, HALF1, tgl16)

        ms, ss = c1[:U], c1[U:]
        m16 = ms[0]
        s16 = ss[0]
        for u in range(1, U):
            m16 = jnp.maximum(m16, ms[u])
            s16 = s16 + ss[u]

        ma = jnp.max(m16)
        tg = jnp.max(tgl16)

        def rare_scan():
            bm = jnp.full((L,), ma, jnp.float32)

            def eq_scan(buf, goff, steps, mi0):
                def rstep(i, mi):
                    for u in range(U):
                        off = i * (L * U) + u * L
                        v = buf[pl.ds(off, L)]
                        gidx = goff + off + lane
                        hit = (v == bm) & (mi == BIG)
                        mi = jnp.where(hit, gidx, mi)
                    return mi
                return lax.fori_loop(0, steps, rstep, mi0)

            @pl.when(r < RPW - 1)
            def _():
                pltpu.make_async_copy(
                    x_hbm.at[row + 1].at[pl.ds(0, HALF0)], buf0, sem0).wait()

            pltpu.sync_copy(x_hbm.at[row].at[pl.ds(0, HALF0)], buf0)
            mi16 = eq_scan(buf0, 0, STEPS0, jnp.full((L,), BIG, jnp.int32))
            mi16 = eq_scan(buf1, HALF0, STEPS1, mi16)

            @pl.when(r < RPW - 1)
            def _():
                pltpu.make_async_copy(
                    x_hbm.at[row + 1].at[pl.ds(0, HALF0)], buf0, sem0).start()

            return mi16

        mi16 = lax.cond(tg == ma, rare_scan,
                        lambda: jnp.full((L,), BIG, jnp.int32))

        om[pl.ds(r * L, L)] = m16
        os_[pl.ds(r * L, L)] = s16
        omi[pl.ds(r * L, L)] = mi16
        otgl[pl.ds(r * L, L)] = tgl16
        otv[pl.ds(r * L, L)] = t16
        return 0

    lax.fori_loop(0, RPW, row_step, 0)

    fb = base * L
    pltpu.sync_copy(om, m_hbm.at[pl.ds(fb, RPW * L)])
    pltpu.sync_copy(os_, s_hbm.at[pl.ds(fb, RPW * L)])
    pltpu.sync_copy(omi, mi_hbm.at[pl.ds(fb, RPW * L)])
    pltpu.sync_copy(otgl, tgl_hbm.at[pl.ds(fb, RPW * L)])
    pltpu.sync_copy(otv, tv_hbm.at[pl.ds(fb, RPW * L)])


_sc_call = functools.partial(
    pl.kernel,
    out_type=(jax.ShapeDtypeStruct((RSC * L,), jnp.float32),
              jax.ShapeDtypeStruct((RSC * L,), jnp.float32),
              jax.ShapeDtypeStruct((RSC * L,), jnp.int32),
              jax.ShapeDtypeStruct((RSC * L,), jnp.float32),
              jax.ShapeDtypeStruct((RSC * L,), jnp.int32)),
    mesh=plsc.VectorSubcoreMesh(
        core_axis_name="c", subcore_axis_name="s",
        num_cores=NC, num_subcores=NS),
    compiler_params=pltpu.CompilerParams(needs_layout_passes=False),
    scratch_types=[
        pltpu.VMEM((HALF0,), jnp.float32),
        pltpu.VMEM((HALF1,), jnp.float32),
        pltpu.VMEM((RSC,), jnp.int32),
        pltpu.VMEM((RPW * L,), jnp.float32),
        pltpu.VMEM((RPW * L,), jnp.float32),
        pltpu.VMEM((RPW * L,), jnp.int32),
        pltpu.VMEM((RPW * L,), jnp.float32),
        pltpu.VMEM((RPW * L,), jnp.int32),
        pltpu.SemaphoreType.DMA,
        pltpu.SemaphoreType.DMA,
    ],
)(_sc_body)


def _tc_body(tgt_ref, x_ref, o_ref):
    pid = pl.program_id(0)
    iota = lax.broadcasted_iota(jnp.int32, (TCB, 128), 1)

    def tree(op, xs):
        while len(xs) > 1:
            nxt = [op(xs[k], xs[k + 1]) for k in range(0, len(xs) - 1, 2)]
            if len(xs) % 2:
                nxt.append(xs[-1])
            xs = nxt
        return xs[0]

    def fold(base, width, m, s):
        v = x_ref[:, pl.ds(base, width)]
        vs = [v[:, k * 128:(k + 1) * 128] for k in range(width // 128)]
        m = jnp.maximum(m, tree(jnp.maximum, vs))
        s = s + tree(jnp.add, [jnp.exp(u) for u in vs])
        return m, s

    def chunk(i, carry):
        return fold(i * (2 * CH), 2 * CH, *carry)

    m, s = lax.fori_loop(
        0, NCH // 2, chunk,
        (jnp.full((TCB, 128), -jnp.inf, jnp.float32),
         jnp.zeros((TCB, 128), jnp.float32)))
    m, s = fold((NCH - 1) * CH, CH, m, s)

    v = x_ref[:, pl.ds(CTAIL0, 128)]
    live = iota >= 96
    s = s + jnp.where(live, jnp.exp(v), 0.0)
    m = jnp.maximum(m, jnp.where(live, v, -jnp.inf))

    mrow = jnp.max(m, axis=1, keepdims=True)
    srow = jnp.sum(s, axis=1, keepdims=True)

    tg_scalars = [tgt_ref[RSC + pid * TCB + r] for r in range(TCB)]
    lane1 = lax.broadcasted_iota(jnp.int32, (1, 128), 1)

    def _pick(r):
        t = tg_scalars[r]
        off = pl.multiple_of((t >> 7) << 7, 128)
        v = x_ref[pl.ds(r, 1), pl.ds(off, 128)]
        return jnp.sum(jnp.where(lane1 == (t & 127), v, 0.0),
                       axis=1, keepdims=True)

    tglv = jnp.concatenate([_pick(r) for r in range(TCB)], axis=0)
    trow = jnp.concatenate(
        [jnp.full((1, 1), t, jnp.int32) for t in tg_scalars], axis=0)

    anytie = jnp.any(tglv == mrow)

    def tie_scan():
        def body(i, mi):
            base = i * CH
            for j in range(11):
                v = x_ref[:, pl.ds(base + j * 128, 128)]
                colv = iota + (base + j * 128)
                hit = jnp.logical_and(v == mrow, colv < mi)
                mi = jnp.where(hit, colv, mi)
            return mi
        mi = lax.fori_loop(0, NCH, body,
                           jnp.full((TCB, 128), BIG, jnp.int32))
        v = x_ref[:, pl.ds(CTAIL0, 128)]
        colv = iota + CTAIL0
        hit = jnp.logical_and(jnp.logical_and(live, v == mrow), colv < mi)
        mi = jnp.where(hit, colv, mi)
        return jnp.min(mi, axis=1, keepdims=True)

    mirow = lax.cond(anytie, tie_scan,
                     lambda: jnp.full((TCB, 1), BIG, jnp.int32))

    correct = jnp.logical_and(tglv == mrow, mirow == trow)
    wrong = 1.0 - correct.astype(jnp.float32)
    bsum = jnp.sum((jnp.log(srow) - tglv) * wrong)

    @pl.when(pid == 0)
    def _():
        o_ref[0, 0] = 0.0
    o_ref[0, 0] += bsum


_tc_partial = pl.pallas_call(
    _tc_body,
    grid_spec=pltpu.PrefetchScalarGridSpec(
        num_scalar_prefetch=1,
        grid=(RTC // TCB,),
        in_specs=[pl.BlockSpec((TCB, C), lambda i, *_: (RSC // TCB + i, 0))],
        out_specs=pl.BlockSpec(memory_space=pltpu.SMEM),
    ),
    out_shape=jax.ShapeDtypeStruct((1, 1), jnp.float32),
)


def _combine_body(m_ref, s_ref, mi_ref, tgl_ref, tv_ref, tc_ref, o_ref):
    m = m_ref[...]
    mrow = jnp.max(m, axis=1, keepdims=True)
    srow = jnp.sum(s_ref[...], axis=1, keepdims=True)
    mifirst = jnp.min(mi_ref[...], axis=1, keepdims=True)
    tgl = tgl_ref[:, 0:1]
    tv = tv_ref[:, 0:1]
    correct = jnp.logical_and(tgl == mrow, mifirst == tv)
    wrong = 1.0 - correct.astype(jnp.float32)
    loss = (jnp.log(srow) - tgl) * wrong
    o_ref[0, 0] = (jnp.sum(loss) + tc_ref[0, 0]) / jnp.float32(R)


_combine = pl.pallas_call(
    _combine_body,
    out_shape=jax.ShapeDtypeStruct((1, 1), jnp.float32),
    in_specs=[pl.BlockSpec(memory_space=pltpu.VMEM)] * 5
    + [pl.BlockSpec(memory_space=pltpu.SMEM)],
    out_specs=pl.BlockSpec(memory_space=pltpu.SMEM),
)


def kernel(output, target):
    x = output.reshape(R, C)
    t = target.reshape(R).astype(jnp.int32)
    m, s, mi, tgl, tv = _sc_call(x, t)
    tc_sum = _tc_partial(t, x)
    return _combine(m.reshape(RSC, L), s.reshape(RSC, L),
                    mi.reshape(RSC, L), tgl.reshape(RSC, L),
                    tv.reshape(RSC, L), tc_sum).reshape(())
